# Initial kernel scaffold; baseline (speedup 1.0000x reference)
import jax
import jax.numpy as jnp
from jax.experimental import pallas as pl


def kernel(x, embed_table, W, b):
    raise NotImplementedError("write your pallas kernel here")


# speedup vs baseline: 1.1230x; 1.1230x over previous
import functools

import jax
import jax.numpy as jnp
from jax import lax
from jax.experimental import pallas as pl
from jax.experimental.pallas import tpu as pltpu
from jax.experimental.pallas import tpu_sc as plsc

V = 1000
VP = 1024
C = 8
B = 4096

NC = 2
NS = 16
NW = NC * NS
BPW = B // NW
CB = 4
RB = CB * C
NCH = BPW // CB
XN = BPW * C
LANES = 16


def _mm_body(emb_ref, w_ref, m_ref):
    m_ref[0] = lax.dot_general(
        emb_ref[...], w_ref[0],
        (((1,), (1,)), ((), ())),
        preferred_element_type=jnp.float32)


def _precompute(emb_p, wr_p):
    return pl.pallas_call(
        _mm_body,
        grid=(C,),
        in_specs=[
            pl.BlockSpec((VP, VP), lambda c: (0, 0)),
            pl.BlockSpec((1, VP, VP), lambda c: (c, 0, 0)),
        ],
        out_specs=pl.BlockSpec((1, VP, VP), lambda c: (c, 0, 0)),
        out_shape=jax.ShapeDtypeStruct((C, VP, VP), jnp.float32),
    )(emb_p, wr_p)


_MESH = plsc.VectorSubcoreMesh(core_axis_name="c", subcore_axis_name="s")


@functools.partial(
    pl.kernel,
    mesh=_MESH,
    out_type=jax.ShapeDtypeStruct((B, VP), jnp.float32),
    scratch_types=[
        pltpu.VMEM((XN,), jnp.int32),
        pltpu.VMEM((XN,), jnp.int32),
        pltpu.VMEM((2, RB, VP), jnp.float32),
        pltpu.VMEM((2, CB, VP), jnp.float32),
        pltpu.VMEM((VP,), jnp.float32),
        pltpu.SemaphoreType.DMA,
        pltpu.SemaphoreType.DMA,
        pltpu.SemaphoreType.DMA,
        pltpu.SemaphoreType.DMA,
    ],
)
def _sc_gather(m_hbm, x_hbm, bias_hbm, out_hbm,
               x_v, idx_v, rows_v, out_v, bias_v,
               gsem0, gsem1, osem0, osem1):
    gsems = (gsem0, gsem1)
    osems = (osem0, osem1)
    wid = lax.axis_index("s") * NC + lax.axis_index("c")
    base_b = wid * BPW
    base_i = wid * XN

    pltpu.sync_copy(x_hbm.at[pl.ds(base_i, XN)], x_v)
    pltpu.sync_copy(bias_hbm, bias_v)

    offs = jnp.bitwise_and(lax.iota(jnp.int32, LANES), C - 1) * VP

    @pl.loop(0, XN, step=LANES)
    def _(i):
        idx_v[pl.ds(i, LANES)] = x_v[pl.ds(i, LANES)] + offs

    def gather_copy(g, k):
        start = pl.multiple_of(g * RB, 8)
        return pltpu.make_async_copy(
            m_hbm.at[idx_v.at[pl.ds(start, RB)]], rows_v.at[k], gsems[k])

    def out_copy(g, k):
        return pltpu.make_async_copy(
            out_v.at[k], out_hbm.at[pl.ds(base_b + g * CB, CB)], osems[k])

    def compute(k):
        for e in range(CB):
            @pl.loop(0, VP, step=4 * LANES)
            def _(v):
                for u in range(4):
                    s = pl.ds(v + u * LANES, LANES)
                    acc = rows_v[k, e * C, s]
                    for r in range(1, C):
                        acc = acc + rows_v[k, e * C + r, s]
                    t = acc + bias_v[s]
                    a = jnp.abs(t)
                    ex = jnp.exp(a + a)
                    pos = 1.0 - 2.0 / (ex + 1.0)
                    out_v[k, e, s] = jnp.where(t < 0.0, -pos, pos)

    gather_copy(0, 0).start()
    gather_copy(1, 1).start()

    @pl.loop(0, NCH, step=2)
    def _(g):
        for k in range(2):
            gg = g + k
            gather_copy(gg, k).wait()

            @pl.when(gg >= 2)
            def _():
                out_copy(gg - 2, k).wait()

            compute(k)
            out_copy(gg, k).start()

            @pl.when(gg + 2 < NCH)
            def _():
                gather_copy(gg + 2, k).start()

    for k in range(2):
        out_copy(NCH - 2 + k, k).wait()


def kernel(x, embed_table, W, b):
    emb_p = jnp.pad(embed_table, ((0, VP - V), (0, VP - V)))
    wr = W.reshape(V, C, V).transpose(1, 0, 2)
    wr_p = jnp.pad(wr, ((0, 0), (0, VP - V), (0, VP - V)))
    b_p = jnp.pad(b, (0, VP - V))
    m = _precompute(emb_p, wr_p).reshape(C * VP, VP)
    out_p = _sc_gather(m, x.reshape(-1), b_p)
    return out_p[:, :V]


# speedup vs baseline: 1.4827x; 1.3203x over previous
import functools

import jax
import jax.numpy as jnp
from jax import lax
from jax.experimental import pallas as pl
from jax.experimental.pallas import tpu as pltpu
from jax.experimental.pallas import tpu_sc as plsc

V = 1000
VP = 1024
C = 8
B = 4096

NC = 2
NS = 16
NW = NC * NS
BPW = B // NW
CB = 4
RB = CB * C
NCH = BPW // CB
XN = BPW * C
LANES = 16
SUB = 8


def _mm_body(emb_ref, w_ref, m_ref):
    m_ref[0] = lax.dot_general(
        emb_ref[...], w_ref[0],
        (((1,), (1,)), ((), ())),
        preferred_element_type=jnp.float32)


def _precompute(emb_p, wr_p):
    return pl.pallas_call(
        _mm_body,
        grid=(C,),
        in_specs=[
            pl.BlockSpec((VP, VP), lambda c: (0, 0)),
            pl.BlockSpec((1, VP, VP), lambda c: (c, 0, 0)),
        ],
        out_specs=pl.BlockSpec((1, VP, VP), lambda c: (c, 0, 0)),
        out_shape=jax.ShapeDtypeStruct((C, VP, VP), jnp.float32),
    )(emb_p, wr_p)


_MESH = plsc.VectorSubcoreMesh(core_axis_name="c", subcore_axis_name="s")


@functools.partial(
    pl.kernel,
    mesh=_MESH,
    out_type=jax.ShapeDtypeStruct((B, SUB, 128), jnp.float32),
    scratch_types=[
        pltpu.VMEM((XN,), jnp.int32),
        pltpu.VMEM((XN,), jnp.int32),
        pltpu.VMEM((2, RB, SUB, 128), jnp.float32),
        pltpu.VMEM((2, CB, SUB, 128), jnp.float32),
        pltpu.VMEM((SUB, 128), jnp.float32),
        pltpu.SemaphoreType.DMA,
        pltpu.SemaphoreType.DMA,
        pltpu.SemaphoreType.DMA,
        pltpu.SemaphoreType.DMA,
    ],
)
def _sc_gather(m_hbm, x_hbm, bias_hbm, out_hbm,
               x_v, idx_v, rows_v, out_v, bias_v,
               gsem0, gsem1, osem0, osem1):
    gsems = (gsem0, gsem1)
    osems = (osem0, osem1)
    wid = lax.axis_index("s") * NC + lax.axis_index("c")
    base_b = wid * BPW
    base_i = wid * XN

    pltpu.sync_copy(x_hbm.at[pl.ds(base_i, XN)], x_v)
    pltpu.sync_copy(bias_hbm, bias_v)

    offs = jnp.bitwise_and(lax.iota(jnp.int32, LANES), C - 1) * VP

    @pl.loop(0, XN, step=LANES)
    def _(i):
        idx_v[pl.ds(i, LANES)] = x_v[pl.ds(i, LANES)] + offs

    def gather_copy(g, k):
        start = pl.multiple_of(g * RB, 8)
        return pltpu.make_async_copy(
            m_hbm.at[idx_v.at[pl.ds(start, RB)]], rows_v.at[k], gsems[k])

    def out_copy(g, k):
        return pltpu.make_async_copy(
            out_v.at[k], out_hbm.at[pl.ds(base_b + g * CB, CB)], osems[k])

    def compute(k):
        for e in range(CB):
            @pl.loop(0, SUB)
            def _(sub):
                for u in range(128 // LANES):
                    s = pl.ds(u * LANES, LANES)
                    acc = rows_v[k, e * C, sub, s]
                    for r in range(1, C):
                        acc = acc + rows_v[k, e * C + r, sub, s]
                    t = acc + bias_v[sub, s]
                    a = jnp.abs(t)
                    ex = jnp.exp(a + a)
                    pos = 1.0 - 2.0 / (ex + 1.0)
                    out_v[k, e, sub, s] = jnp.where(t < 0.0, -pos, pos)

    gather_copy(0, 0).start()
    gather_copy(1, 1).start()

    @pl.loop(0, NCH, step=2)
    def _(g):
        for k in range(2):
            gg = g + k
            gather_copy(gg, k).wait()

            @pl.when(gg >= 2)
            def _():
                out_copy(gg - 2, k).wait()

            compute(k)
            out_copy(gg, k).start()

            @pl.when(gg + 2 < NCH)
            def _():
                gather_copy(gg + 2, k).start()

    for k in range(2):
        out_copy(NCH - 2 + k, k).wait()


def kernel(x, embed_table, W, b):
    emb_p = jnp.pad(embed_table, ((0, VP - V), (0, VP - V)))
    wr = W.reshape(V, C, V).transpose(1, 0, 2)
    wr_p = jnp.pad(wr, ((0, 0), (0, VP - V), (0, VP - V)))
    b_p = jnp.pad(b, (0, VP - V)).reshape(SUB, 128)
    m = _precompute(emb_p, wr_p).reshape(C * VP, SUB, 128)
    out_p = _sc_gather(m, x.reshape(-1), b_p)
    return out_p.reshape(B, VP)[:, :V]


# speedup vs baseline: 2.2950x; 1.5479x over previous
import functools

import jax
import jax.numpy as jnp
from jax import lax
from jax.experimental import pallas as pl
from jax.experimental.pallas import tpu as pltpu
from jax.experimental.pallas import tpu_sc as plsc

V = 1000
VP = 1024
C = 8
B = 4096

NC = 2
NS = 16
NW = NC * NS
BPW = B // NW
CB = 4
RB = CB * C
NCH = BPW // CB
XN = BPW * C
LANES = 16
SUB = 8


def _mm_body(embv_ref, w_ref, m_ref):
    h = pl.program_id(1)
    part = lax.dot_general(
        embv_ref[0, 0], w_ref[...],
        (((1,), (1,)), ((), ())),
        preferred_element_type=jnp.float32)

    @pl.when(h == 0)
    def _():
        m_ref[0] = part

    @pl.when(h == 1)
    def _():
        m_ref[0] += part


def _precompute(embv, w_p):
    return pl.pallas_call(
        _mm_body,
        grid=(C, 2),
        in_specs=[
            pl.BlockSpec((1, 1, VP, VP), lambda c, h: (c, h, 0, 0)),
            pl.BlockSpec((VP, VP), lambda c, h: (0, (1000 * c) // VP + h)),
        ],
        out_specs=pl.BlockSpec((1, VP, VP), lambda c, h: (c, 0, 0)),
        out_shape=jax.ShapeDtypeStruct((C, VP, VP), jnp.float32),
    )(embv, w_p)


def _emb_variants(embed_table):
    variants = []
    for c in range(C):
        for h in range(2):
            j = (1000 * c) // VP + h
            s = VP * j - V * c
            lo, hi = max(0, -s), min(VP, V - s)
            if hi <= lo:
                variants.append(jnp.zeros((VP, VP), jnp.float32))
            else:
                blk = embed_table[:, lo + s:hi + s]
                variants.append(jnp.pad(blk, ((0, VP - V), (lo, VP - hi))))
    return jnp.stack(variants).reshape(C, 2, VP, VP)


_MESH = plsc.VectorSubcoreMesh(core_axis_name="c", subcore_axis_name="s")


@functools.partial(
    pl.kernel,
    mesh=_MESH,
    out_type=jax.ShapeDtypeStruct((B, SUB, 128), jnp.float32),
    scratch_types=[
        pltpu.VMEM((XN // 128, 128), jnp.int32),
        pltpu.VMEM((XN,), jnp.int32),
        pltpu.VMEM((2, RB, SUB, 128), jnp.float32),
        pltpu.VMEM((2, CB, SUB, 128), jnp.float32),
        pltpu.VMEM((SUB, 128), jnp.float32),
        pltpu.SemaphoreType.DMA,
        pltpu.SemaphoreType.DMA,
        pltpu.SemaphoreType.DMA,
        pltpu.SemaphoreType.DMA,
    ],
)
def _sc_gather(m_hbm, x_hbm, bias_hbm, out_hbm,
               x_v, idx_v, rows_v, out_v, bias_v,
               gsem0, gsem1, osem0, osem1):
    gsems = (gsem0, gsem1)
    osems = (osem0, osem1)
    wid = lax.axis_index("s") * NC + lax.axis_index("c")
    base_b = wid * BPW

    pltpu.sync_copy(x_hbm.at[pl.ds(wid * (XN // 128), XN // 128)], x_v)
    pltpu.sync_copy(bias_hbm, bias_v)

    offs = jnp.bitwise_and(lax.iota(jnp.int32, LANES), C - 1) * VP

    for row in range(XN // 128):
        @pl.loop(0, 128, step=LANES)
        def _(i):
            idx_v[pl.ds(row * 128 + i, LANES)] = x_v[row, pl.ds(i, LANES)] + offs

    def gather_copy(g, k):
        start = pl.multiple_of(g * RB, 8)
        return pltpu.make_async_copy(
            m_hbm.at[idx_v.at[pl.ds(start, RB)]], rows_v.at[k], gsems[k])

    def out_copy(g, k):
        return pltpu.make_async_copy(
            out_v.at[k], out_hbm.at[pl.ds(base_b + g * CB, CB)], osems[k])

    def compute(k):
        for e in range(CB):
            @pl.loop(0, SUB)
            def _(sub):
                for u in range(128 // LANES):
                    s = pl.ds(u * LANES, LANES)
                    acc = rows_v[k, e * C, sub, s]
                    for r in range(1, C):
                        acc = acc + rows_v[k, e * C + r, sub, s]
                    t = acc + bias_v[sub, s]
                    a = jnp.abs(t)
                    ex = jnp.exp(a + a)
                    pos = 1.0 - 2.0 / (ex + 1.0)
                    out_v[k, e, sub, s] = jnp.where(t < 0.0, -pos, pos)

    gather_copy(0, 0).start()
    gather_copy(1, 1).start()

    @pl.loop(0, NCH, step=2)
    def _(g):
        for k in range(2):
            gg = g + k
            gather_copy(gg, k).wait()

            @pl.when(gg >= 2)
            def _():
                out_copy(gg - 2, k).wait()

            compute(k)
            out_copy(gg, k).start()

            @pl.when(gg + 2 < NCH)
            def _():
                gather_copy(gg + 2, k).start()

    for k in range(2):
        out_copy(NCH - 2 + k, k).wait()


def kernel(x, embed_table, W, b):
    embv = _emb_variants(embed_table)
    w_p = jnp.pad(W, ((0, VP - V), (0, C * VP - C * V)))
    b_p = jnp.pad(b, (0, VP - V)).reshape(SUB, 128)
    m = _precompute(embv, w_p).reshape(C * VP, SUB, 128)
    out_p = _sc_gather(m, x.reshape(B * C // 128, 128), b_p)
    return out_p.reshape(B, VP)[:, :V]


# speedup vs baseline: 2.5764x; 1.1226x over previous
import functools

import jax
import jax.numpy as jnp
from jax import lax
from jax.experimental import pallas as pl
from jax.experimental.pallas import tpu as pltpu
from jax.experimental.pallas import tpu_sc as plsc

V = 1000
VP = 1024
C = 8
B = 4096

NC = 2
NS = 16
NW = NC * NS
BPW = B // NW
CB = 4
RB = CB * C
NCH = BPW // CB
XN = BPW * C
LANES = 16
SUB = 8


KB = 512
KW = 3 * KB


def _mm_body(embv_ref, w_ref, m_ref):
    kk = pl.program_id(1)
    part = lax.dot_general(
        embv_ref[0], w_ref[...],
        (((1,), (1,)), ((), ())),
        preferred_element_type=jnp.float32)

    @pl.when(kk == 0)
    def _():
        m_ref[0] = part

    @pl.when(kk != 0)
    def _():
        m_ref[0] += part


def _precompute(embv, w_p):
    return pl.pallas_call(
        _mm_body,
        grid=(C, KW // KB),
        in_specs=[
            pl.BlockSpec((1, VP, KB), lambda c, kk: (c, 0, kk)),
            pl.BlockSpec((VP, KB), lambda c, kk: (0, 1 + 2 * c + kk)),
        ],
        out_specs=pl.BlockSpec((1, VP, VP), lambda c, kk: (c, 0, 0)),
        out_shape=jax.ShapeDtypeStruct((C, VP, VP), jnp.float32),
    )(embv, w_p)


def _emb_variants(embed_table):
    variants = []
    for c in range(C):
        lo = KB - 24 * c
        variants.append(jnp.pad(embed_table, ((0, VP - V), (lo, KW - V - lo))))
    return jnp.stack(variants)


_MESH = plsc.VectorSubcoreMesh(core_axis_name="c", subcore_axis_name="s")


@functools.partial(
    pl.kernel,
    mesh=_MESH,
    out_type=jax.ShapeDtypeStruct((B, SUB, 128), jnp.float32),
    scratch_types=[
        pltpu.VMEM((XN // 128, 128), jnp.int32),
        pltpu.VMEM((XN,), jnp.int32),
        pltpu.VMEM((2, RB, SUB, 128), jnp.float32),
        pltpu.VMEM((2, CB, SUB, 128), jnp.float32),
        pltpu.VMEM((SUB, 128), jnp.float32),
        pltpu.SemaphoreType.DMA,
        pltpu.SemaphoreType.DMA,
        pltpu.SemaphoreType.DMA,
        pltpu.SemaphoreType.DMA,
    ],
)
def _sc_gather(m_hbm, x_hbm, bias_hbm, out_hbm,
               x_v, idx_v, rows_v, out_v, bias_v,
               gsem0, gsem1, osem0, osem1):
    gsems = (gsem0, gsem1)
    osems = (osem0, osem1)
    wid = lax.axis_index("s") * NC + lax.axis_index("c")
    base_b = wid * BPW

    pltpu.sync_copy(x_hbm.at[pl.ds(wid * (XN // 128), XN // 128)], x_v)
    pltpu.sync_copy(bias_hbm, bias_v)

    offs = jnp.bitwise_and(lax.iota(jnp.int32, LANES), C - 1) * VP

    for row in range(XN // 128):
        @pl.loop(0, 128, step=LANES)
        def _(i):
            idx_v[pl.ds(row * 128 + i, LANES)] = x_v[row, pl.ds(i, LANES)] + offs

    def gather_copy(g, k):
        start = pl.multiple_of(g * RB, 8)
        return pltpu.make_async_copy(
            m_hbm.at[idx_v.at[pl.ds(start, RB)]], rows_v.at[k], gsems[k])

    def out_copy(g, k):
        return pltpu.make_async_copy(
            out_v.at[k], out_hbm.at[pl.ds(base_b + g * CB, CB)], osems[k])

    def compute(k):
        for e in range(CB):
            @pl.loop(0, SUB)
            def _(sub):
                for u in range(128 // LANES):
                    s = pl.ds(u * LANES, LANES)
                    acc = rows_v[k, e * C, sub, s]
                    for r in range(1, C):
                        acc = acc + rows_v[k, e * C + r, sub, s]
                    t = acc + bias_v[sub, s]
                    a = jnp.abs(t)
                    ex = jnp.exp(a + a)
                    pos = 1.0 - 2.0 / (ex + 1.0)
                    out_v[k, e, sub, s] = jnp.where(t < 0.0, -pos, pos)

    gather_copy(0, 0).start()
    gather_copy(1, 1).start()

    @pl.loop(0, NCH, step=2)
    def _(g):
        for k in range(2):
            gg = g + k
            gather_copy(gg, k).wait()

            @pl.when(gg >= 2)
            def _():
                out_copy(gg - 2, k).wait()

            compute(k)
            out_copy(gg, k).start()

            @pl.when(gg + 2 < NCH)
            def _():
                gather_copy(gg + 2, k).start()

    for k in range(2):
        out_copy(NCH - 2 + k, k).wait()


def kernel(x, embed_table, W, b):
    embv = _emb_variants(embed_table)
    w_p = jnp.pad(W, ((0, VP - V), (VP, KB + VP * (C - 1) + KW - VP - C * V)))
    b_p = jnp.pad(b, (0, VP - V)).reshape(SUB, 128)
    m = _precompute(embv, w_p).reshape(C * VP, SUB, 128)
    out_p = _sc_gather(m, x.reshape(B * C // 128, 128), b_p)
    return out_p.reshape(B, VP)[:, :V]
